# baseline (speedup 1.0000x reference)
import functools

import jax
import jax.numpy as jnp
from jax import lax
from jax.experimental import pallas as pl
from jax.experimental.pallas import tpu as pltpu
from jax.experimental.pallas import tpu_sc as plsc

LANES = 16
NUM_CORES = 2
NUM_SUBCORES = 16
NUM_WORKERS = NUM_CORES * NUM_SUBCORES


def _sc_add_posemb(x, embed_weight, off_arr, *, tc):
    B, T, D = x.shape
    max_len = embed_weight.shape[0]
    n_chunks = T // NUM_WORKERS // tc
    vregs_per_row = D // LANES
    emb_rows = tc + 8

    mesh = plsc.VectorSubcoreMesh(core_axis_name="c", subcore_axis_name="s")

    @functools.partial(
        pl.kernel,
        mesh=mesh,
        out_type=jax.ShapeDtypeStruct((B, T, D), jnp.float32),
        scratch_types=[
            pltpu.VMEM((emb_rows, D), jnp.float32),
            pltpu.VMEM((B, tc, D), jnp.float32),
            pltpu.VMEM((LANES,), jnp.int32),
        ],
    )
    def body(x_hbm, emb_hbm, off_hbm, out_hbm, emb_v, x_v, off_v):
        wid = lax.axis_index("s") * NUM_CORES + lax.axis_index("c")
        pltpu.sync_copy(off_hbm, off_v)
        offset = off_v[pl.ds(0, LANES)][0]
        base = wid * (T // NUM_WORKERS)

        def chunk_body(c, carry):
            t0 = pl.multiple_of(base + c * tc, 8)
            start = offset + t0
            al = jnp.minimum((start // 8) * 8, max_len - emb_rows)
            al = pl.multiple_of(al, 8)
            rem = start - al
            pltpu.sync_copy(emb_hbm.at[pl.ds(al, emb_rows)], emb_v)
            for b in range(B):
                pltpu.sync_copy(x_hbm.at[b, pl.ds(t0, tc)], x_v.at[b])

            def row_body(r, carry2):
                def vreg_body(j, carry3):
                    e = emb_v[r + rem, pl.ds(j * LANES, LANES)]
                    for b in range(B):
                        plsc.addupdate(x_v.at[b, r, pl.ds(j * LANES, LANES)], e)
                    return carry3

                return lax.fori_loop(0, vregs_per_row, vreg_body, carry2)

            lax.fori_loop(0, tc, row_body, 0)

            for b in range(B):
                pltpu.sync_copy(x_v.at[b], out_hbm.at[b, pl.ds(t0, tc)])
            return carry

        lax.fori_loop(0, n_chunks, chunk_body, 0)

    return body(x, embed_weight, off_arr)


def kernel(x, embed_weight, offset):
    off_arr = jnp.full((LANES,), offset, dtype=jnp.int32)
    return _sc_add_posemb(x, embed_weight, off_arr, tc=8)


# speedup vs baseline: 1.1178x; 1.1178x over previous
import functools

import jax
import jax.numpy as jnp
from jax import lax
from jax.experimental import pallas as pl
from jax.experimental.pallas import tpu as pltpu
from jax.experimental.pallas import tpu_sc as plsc

LANES = 16
NUM_CORES = 2
NUM_SUBCORES = 16
NUM_WORKERS = NUM_CORES * NUM_SUBCORES


def _sc_add_posemb(x, embed_weight, off_arr, *, tc):
    B, T, D = x.shape
    max_len = embed_weight.shape[0]
    n_chunks = T // NUM_WORKERS // tc
    vregs_per_row = D // LANES
    emb_rows = tc + 8

    mesh = plsc.VectorSubcoreMesh(core_axis_name="c", subcore_axis_name="s")

    @functools.partial(
        pl.kernel,
        mesh=mesh,
        out_type=jax.ShapeDtypeStruct((B, T, D), jnp.float32),
        scratch_types=[
            pltpu.VMEM((emb_rows, D), jnp.float32),
            pltpu.VMEM((B, tc, D), jnp.float32),
            pltpu.VMEM((LANES,), jnp.int32),
        ],
    )
    def body(x_hbm, emb_hbm, off_hbm, out_hbm, emb_v, x_v, off_v):
        wid = lax.axis_index("s") * NUM_CORES + lax.axis_index("c")
        pltpu.sync_copy(off_hbm, off_v)
        offset = off_v[pl.ds(0, LANES)][0]
        base = wid * (T // NUM_WORKERS)

        def chunk_body(c, carry):
            t0 = pl.multiple_of(base + c * tc, 8)
            start = offset + t0
            al = jnp.minimum((start // 8) * 8, max_len - emb_rows)
            al = pl.multiple_of(al, 8)
            rem = start - al
            pltpu.sync_copy(emb_hbm.at[pl.ds(al, emb_rows)], emb_v)
            for b in range(B):
                pltpu.sync_copy(x_hbm.at[b, pl.ds(t0, tc)], x_v.at[b])

            def row_body(r, carry2):
                @plsc.parallel_loop(0, vregs_per_row, unroll=8)
                def vreg_body(j):
                    e = emb_v[r + rem, pl.ds(j * LANES, LANES)]
                    for b in range(B):
                        plsc.addupdate(x_v.at[b, r, pl.ds(j * LANES, LANES)], e)

                return carry2

            lax.fori_loop(0, tc, row_body, 0)

            for b in range(B):
                pltpu.sync_copy(x_v.at[b], out_hbm.at[b, pl.ds(t0, tc)])
            return carry

        lax.fori_loop(0, n_chunks, chunk_body, 0)

    return body(x, embed_weight, off_arr)


def kernel(x, embed_weight, offset):
    off_arr = jnp.full((LANES,), offset, dtype=jnp.int32)
    return _sc_add_posemb(x, embed_weight, off_arr, tc=8)


# speedup vs baseline: 1.9845x; 1.7754x over previous
import functools

import jax
import jax.numpy as jnp
from jax import lax
from jax.experimental import pallas as pl
from jax.experimental.pallas import tpu as pltpu
from jax.experimental.pallas import tpu_sc as plsc

LANES = 16
NUM_CORES = 2
NUM_SUBCORES = 16
NUM_WORKERS = NUM_CORES * NUM_SUBCORES
NBUF = 2


def _sc_add_posemb(x, embed_weight, off_arr, *, tc):
    B, T, D = x.shape
    max_len = embed_weight.shape[0]
    rows_per_worker = T // NUM_WORKERS
    n_chunks = rows_per_worker // tc
    vregs_per_row = D // LANES
    emb_rows = tc + 8

    mesh = plsc.VectorSubcoreMesh(core_axis_name="c", subcore_axis_name="s")

    @functools.partial(
        pl.kernel,
        mesh=mesh,
        out_type=jax.ShapeDtypeStruct((B, T, D), jnp.float32),
        scratch_types=(
            [pltpu.VMEM((emb_rows, D), jnp.float32) for _ in range(NBUF)]
            + [pltpu.VMEM((B, tc, D), jnp.float32) for _ in range(NBUF)]
            + [pltpu.VMEM((LANES,), jnp.int32)]
            + [pltpu.SemaphoreType.DMA for _ in range(2 * NBUF)]
        ),
    )
    def body(x_hbm, emb_hbm, off_hbm, out_hbm, *scratch):
        emb_bufs = scratch[:NBUF]
        x_bufs = scratch[NBUF:2 * NBUF]
        off_v = scratch[2 * NBUF]
        lsems = scratch[2 * NBUF + 1:2 * NBUF + 1 + NBUF]
        ssems = scratch[2 * NBUF + 1 + NBUF:]

        wid = lax.axis_index("s") * NUM_CORES + lax.axis_index("c")
        pltpu.sync_copy(off_hbm, off_v)
        offset = off_v[pl.ds(0, LANES)][0]
        base = wid * rows_per_worker

        def chunk_start(c):
            return pl.multiple_of(base + c * tc, 8)

        def emb_align(c):
            start = offset + chunk_start(c)
            al = pl.multiple_of(jnp.minimum((start // 8) * 8, max_len - emb_rows), 8)
            return al, start - al

        def start_load(c):
            s = c % NBUF
            t0 = chunk_start(c)
            al, _ = emb_align(c)
            copies = [pltpu.async_copy(emb_hbm.at[pl.ds(al, emb_rows)],
                                       emb_bufs[s], lsems[s])]
            for b in range(B):
                copies.append(pltpu.async_copy(x_hbm.at[b, pl.ds(t0, tc)],
                                               x_bufs[s].at[b], lsems[s]))
            return copies

        def start_store(c):
            s = c % NBUF
            t0 = chunk_start(c)
            return [pltpu.async_copy(x_bufs[s].at[b],
                                     out_hbm.at[b, pl.ds(t0, tc)], ssems[s])
                    for b in range(B)]

        def compute(c):
            s = c % NBUF
            _, rem = emb_align(c)
            emb_v, x_v = emb_bufs[s], x_bufs[s]

            def row_body(r, carry):
                @plsc.parallel_loop(0, vregs_per_row, unroll=8)
                def vreg_body(j):
                    e = emb_v[r + rem, pl.ds(j * LANES, LANES)]
                    for b in range(B):
                        plsc.addupdate(x_v.at[b, r, pl.ds(j * LANES, LANES)], e)

                return carry

            lax.fori_loop(0, tc, row_body, 0)

        loads = [None] * n_chunks
        stores = [None] * n_chunks
        loads[0] = start_load(0)
        for c in range(n_chunks):
            if c + 1 < n_chunks:
                if c - 1 >= 0:
                    for h in stores[c - 1]:
                        h.wait()
                loads[c + 1] = start_load(c + 1)
            for h in loads[c]:
                h.wait()
            compute(c)
            stores[c] = start_store(c)
        for c in range(max(0, n_chunks - 2), n_chunks):
            for h in stores[c]:
                h.wait()

    return body(x, embed_weight, off_arr)


def kernel(x, embed_weight, offset):
    off_arr = jnp.full((LANES,), offset, dtype=jnp.int32)
    return _sc_add_posemb(x, embed_weight, off_arr, tc=8)


# speedup vs baseline: 2.1321x; 1.0744x over previous
import functools

import jax
import jax.numpy as jnp
from jax import lax
from jax.experimental import pallas as pl
from jax.experimental.pallas import tpu as pltpu
from jax.experimental.pallas import tpu_sc as plsc

LANES = 16
NUM_CORES = 2
NUM_SUBCORES = 16
NUM_WORKERS = NUM_CORES * NUM_SUBCORES
NBUF = 3


def _sc_add_posemb(x, embed_weight, off_arr, *, tc):
    B, T, D = x.shape
    rows_per_worker = T // NUM_WORKERS
    n_chunks = rows_per_worker // tc
    vregs_per_row = D // LANES

    mesh = plsc.VectorSubcoreMesh(core_axis_name="c", subcore_axis_name="s")

    @functools.partial(
        pl.kernel,
        mesh=mesh,
        out_type=jax.ShapeDtypeStruct((B, T, D), jnp.float32),
        scratch_types=(
            [pltpu.VMEM((tc, D), jnp.float32) for _ in range(NBUF)]
            + [pltpu.VMEM((B, tc, D), jnp.float32) for _ in range(NBUF)]
            + [pltpu.VMEM((rows_per_worker,), jnp.int32)]
            + [pltpu.VMEM((LANES,), jnp.int32)]
            + [pltpu.SemaphoreType.DMA for _ in range(2 * NBUF)]
        ),
    )
    def body(x_hbm, emb_hbm, off_hbm, out_hbm, *scratch):
        emb_bufs = scratch[:NBUF]
        x_bufs = scratch[NBUF:2 * NBUF]
        idx_flat = scratch[2 * NBUF]
        off_v = scratch[2 * NBUF + 1]
        lsems = scratch[2 * NBUF + 2:2 * NBUF + 2 + NBUF]
        ssems = scratch[2 * NBUF + 2 + NBUF:]

        wid = lax.axis_index("s") * NUM_CORES + lax.axis_index("c")
        pltpu.sync_copy(off_hbm, off_v)
        offset = off_v[pl.ds(0, LANES)][0]
        base = wid * rows_per_worker

        for k in range(rows_per_worker // LANES):
            idx_flat[pl.ds(k * LANES, LANES)] = (
                lax.iota(jnp.int32, LANES) + (base + offset + k * LANES)
            )

        def start_load(c):
            s = c % NBUF
            t0 = pl.multiple_of(base + c * tc, 8)
            copies = [pltpu.async_copy(
                emb_hbm.at[idx_flat.at[pl.ds(c * tc, tc)]],
                emb_bufs[s], lsems[s])]
            for b in range(B):
                copies.append(pltpu.async_copy(x_hbm.at[b, pl.ds(t0, tc)],
                                               x_bufs[s].at[b], lsems[s]))
            return copies

        def start_store(c):
            s = c % NBUF
            t0 = pl.multiple_of(base + c * tc, 8)
            return [pltpu.async_copy(x_bufs[s].at[b],
                                     out_hbm.at[b, pl.ds(t0, tc)], ssems[s])
                    for b in range(B)]

        def compute(c):
            s = c % NBUF
            emb_v, x_v = emb_bufs[s], x_bufs[s]

            def row_body(r, carry):
                @plsc.parallel_loop(0, vregs_per_row, unroll=8)
                def vreg_body(j):
                    e = emb_v[r, pl.ds(j * LANES, LANES)]
                    for b in range(B):
                        plsc.addupdate(x_v.at[b, r, pl.ds(j * LANES, LANES)], e)

                return carry

            lax.fori_loop(0, tc, row_body, 0)

        loads = [None] * n_chunks
        stores = [None] * n_chunks
        for c in range(min(NBUF - 1, n_chunks)):
            loads[c] = start_load(c)
        for c in range(n_chunks):
            if c + NBUF - 1 < n_chunks:
                if c - 1 >= 0:
                    for h in stores[c - 1]:
                        h.wait()
                loads[c + NBUF - 1] = start_load(c + NBUF - 1)
            for h in loads[c]:
                h.wait()
            compute(c)
            stores[c] = start_store(c)
        for c in range(max(0, n_chunks - NBUF), n_chunks):
            for h in stores[c]:
                h.wait()

    return body(x, embed_weight, off_arr)


def kernel(x, embed_weight, offset):
    off_arr = jnp.full((LANES,), offset, dtype=jnp.int32)
    return _sc_add_posemb(x, embed_weight, off_arr, tc=8)
